# baseline (speedup 1.0000x reference)
import functools
import jax
import jax.numpy as jnp
from jax import lax
from jax.experimental import pallas as pl
from jax.experimental.pallas import tpu as pltpu
from jax.experimental.pallas import tpu_sc as plsc

_OHEM_RATIO = 0.25
_EPS = 1e-07

_N = 32 * 8192
_K = int(_N * _OHEM_RATIO)
_NS = 16
_PT = _N // _NS
_VPT = _PT // 16
_HB = 512


def _bce_bits(p, t):
    ap = jnp.abs(p)
    e = jnp.exp(-ap)
    s = e / (e + 2.0)
    s2 = s * s
    l1p = 2.0 * s * (1.0 + s2 * (1.0 / 3.0 + s2 * (0.2 + s2 * (1.0 / 7.0 + s2 * (1.0 / 9.0)))))
    loss = jnp.maximum(p, 0.0) - p * t + l1p
    return plsc.bitcast(loss, jnp.int32)


def _body(pred_h, target_h, out_h, hist_h,
          pred_v, targ_v, bits_v, hall_v, big_v, out_v):
    sid = lax.axis_index("s")
    iota = lax.iota(jnp.int32, 16)
    ones_f = jnp.ones((16,), jnp.float32)
    zero_f = jnp.zeros((16,), jnp.float32)

    base = sid * _PT
    pltpu.sync_copy(pred_h.at[pl.ds(base, _PT)], pred_v)
    pltpu.sync_copy(target_h.at[pl.ds(base, _PT)], targ_v)

    def zero_hall():
        for j in range(_HB // 16):
            hall_v[pl.ds(j * 16, 16)] = zero_f

    def merge_scan(r, last):
        pltpu.sync_copy(hall_v, hist_h.at[pl.ds(sid * _HB, _HB)])
        plsc.subcore_barrier()
        pltpu.sync_copy(hist_h, big_v)

        def mstep(t, acc):
            return tuple(
                acc[j] + big_v[pl.ds(t * _HB + j * 16, 16)]
                for j in range(_HB // 16))

        acc = lax.fori_loop(0, _NS, mstep, (zero_f,) * (_HB // 16))
        plsc.subcore_barrier()
        cr = [acc[j].astype(jnp.int32) for j in range(16)]
        sr = list(acc[16:32])
        rsum = [jnp.sum(cr[rr]) for rr in range(16)]
        cums = [jnp.int32(0)] * 17
        for rr in range(15, -1, -1):
            cums[rr] = cums[rr + 1] + rsum[rr]
        jstar = jnp.int32(0)
        above_rows = jnp.int32(0)
        for rr in range(16):
            hit = (cums[rr] >= r) & (cums[rr + 1] < r)
            jstar = jnp.where(hit, rr, jstar)
            above_rows = jnp.where(hit, cums[rr + 1], above_rows)
        v = cr[15]
        vs = sr[15]
        for rr in range(15):
            v = jnp.where(jstar == rr, cr[rr], v)
            vs = jnp.where(jstar == rr, sr[rr], vs)
        sfx = jnp.cumsum(v[::-1])[::-1] + above_rows
        lane = jnp.max(jnp.where(sfx >= r, iota, -1))
        bucket = jstar * 16 + lane
        above_cnt = above_rows + jnp.sum(jnp.where(iota > lane, v, 0))
        above_rows_sum = zero_f
        for rr in range(16):
            above_rows_sum = above_rows_sum + jnp.where(jstar < rr, sr[rr], zero_f)
        above_sum_vec = above_rows_sum + jnp.where(iota > lane, vs, 0.0)
        if last:
            eq_cnt = jnp.sum(jnp.where(iota == lane, v, 0))
            eq_sum_vec = jnp.where(iota == lane, vs, 0.0)
        else:
            eq_cnt = jnp.int32(0)
            eq_sum_vec = zero_f
        return bucket, above_cnt, above_sum_vec, eq_cnt, eq_sum_vec

    zero_hall()

    def bce_step(i, c):
        sl = pl.ds(i * 16, 16)
        b = _bce_bits(pred_v[sl], targ_v[sl])
        bits_v[sl] = b
        key = lax.shift_right_logical(b, 23)
        plsc.addupdate_scatter(hall_v, [key], ones_f)
        plsc.addupdate_scatter(hall_v, [key + 256], plsc.bitcast(b, jnp.float32))
        return c

    lax.fori_loop(0, _VPT, bce_step, 0, unroll=4)

    b0, above_cnt, accS, _, _ = merge_scan(jnp.int32(_K), False)
    r = jnp.int32(_K) - above_cnt
    accC = above_cnt

    def histogram_round(shift, key_mask, prefix_shift, prefix):
        zero_hall()

        def hist_step(i, c):
            b = bits_v[pl.ds(i * 16, 16)]
            pm = lax.shift_right_logical(b, prefix_shift) == prefix
            key = lax.shift_right_logical(b, shift) & key_mask
            plsc.addupdate_scatter(hall_v, [key], ones_f, mask=pm)
            plsc.addupdate_scatter(hall_v, [key + 256],
                                   plsc.bitcast(b, jnp.float32), mask=pm)
            return c

        lax.fori_loop(0, _VPT, hist_step, 0, unroll=4)

    histogram_round(15, 255, 23, b0)
    b1, above_cnt, above_sum, _, _ = merge_scan(r, False)
    r = r - above_cnt
    accC = accC + above_cnt
    accS = accS + above_sum
    p1 = lax.shift_left(b0, 8) | b1

    histogram_round(7, 255, 15, p1)
    b2, above_cnt, above_sum, _, _ = merge_scan(r, False)
    r = r - above_cnt
    accC = accC + above_cnt
    accS = accS + above_sum
    p2 = lax.shift_left(p1, 8) | b2

    histogram_round(0, 127, 7, p2)
    _, above_cnt, above_sum, eq_cnt, eq_sum = merge_scan(r, True)
    total_c = accC + above_cnt + eq_cnt
    s_vec = accS + above_sum + eq_sum
    total_s = jnp.sum(s_vec)

    @pl.when(sid == 0)
    def _():
        sv = jnp.full((16,), total_s)
        cv = jnp.full((16,), total_c).astype(jnp.float32)
        out_v[:] = sv / (cv + _EPS)
        pltpu.sync_copy(out_v, out_h)


@jax.jit
def kernel(pred, target):
    mesh = plsc.VectorSubcoreMesh(
        core_axis_name="c", subcore_axis_name="s", num_cores=1, num_subcores=16)
    f = pl.kernel(
        _body,
        out_type=(jax.ShapeDtypeStruct((16,), jnp.float32),
                  jax.ShapeDtypeStruct((_NS * _HB,), jnp.float32)),
        mesh=mesh,
        compiler_params=pltpu.CompilerParams(needs_layout_passes=False),
        scratch_types=[
            pltpu.VMEM((_PT,), jnp.float32),
            pltpu.VMEM((_PT,), jnp.float32),
            pltpu.VMEM((_PT,), jnp.int32),
            pltpu.VMEM((_HB,), jnp.float32),
            pltpu.VMEM((_NS * _HB,), jnp.float32),
            pltpu.VMEM((16,), jnp.float32),
        ],
    )
    out, _ = f(pred.reshape(_N), target.reshape(_N))
    return out[0]


# speedup vs baseline: 1.7750x; 1.7750x over previous
import functools
import jax
import jax.numpy as jnp
from jax import lax
from jax.experimental import pallas as pl
from jax.experimental.pallas import tpu as pltpu
from jax.experimental.pallas import tpu_sc as plsc

_OHEM_RATIO = 0.25
_EPS = 1e-07

_N = 32 * 8192
_K = int(_N * _OHEM_RATIO)
_NS = 16
_PT = _N // _NS
_VPT = _PT // 16
_HB = 512


def _bce_bits(p, t):
    ap = jnp.abs(p)
    e = jnp.exp(-ap)
    s = e / (e + 2.0)
    s2 = s * s
    l1p = 2.0 * s * (1.0 + s2 * (1.0 / 3.0 + s2 * (0.2 + s2 * (1.0 / 7.0 + s2 * (1.0 / 9.0)))))
    loss = jnp.maximum(p, 0.0) - p * t + l1p
    return plsc.bitcast(loss, jnp.int32)


def _body(pred_h, target_h, out_h, hist_h,
          pred_v, targ_v, bits_v, hall_v, big_v, out_v):
    sid = lax.axis_index("s")
    iota = lax.iota(jnp.int32, 16)
    ones_f = jnp.ones((16,), jnp.float32)
    zero_f = jnp.zeros((16,), jnp.float32)

    base = sid * _PT
    pltpu.sync_copy(pred_h.at[pl.ds(base, _PT)], pred_v)
    pltpu.sync_copy(target_h.at[pl.ds(base, _PT)], targ_v)

    def zero_hall():
        for j in range(_HB // 16):
            hall_v[pl.ds(j * 16, 16)] = zero_f

    def merge_scan(r, last):
        pltpu.sync_copy(hall_v, hist_h.at[pl.ds(sid * _HB, _HB)])
        plsc.subcore_barrier()
        pltpu.sync_copy(hist_h, big_v)

        def mstep(t, acc):
            return tuple(
                acc[j] + big_v[pl.ds(t * _HB + j * 16, 16)]
                for j in range(_HB // 16))

        acc = lax.fori_loop(0, _NS, mstep, (zero_f,) * (_HB // 16))
        plsc.subcore_barrier()
        cr = [acc[j].astype(jnp.int32) for j in range(16)]
        sr = list(acc[16:32])
        rsum = [jnp.sum(cr[rr]) for rr in range(16)]
        cums = [jnp.int32(0)] * 17
        for rr in range(15, -1, -1):
            cums[rr] = cums[rr + 1] + rsum[rr]
        jstar = jnp.int32(0)
        above_rows = jnp.int32(0)
        for rr in range(16):
            hit = (cums[rr] >= r) & (cums[rr + 1] < r)
            jstar = jnp.where(hit, rr, jstar)
            above_rows = jnp.where(hit, cums[rr + 1], above_rows)
        v = cr[15]
        vs = sr[15]
        for rr in range(15):
            v = jnp.where(jstar == rr, cr[rr], v)
            vs = jnp.where(jstar == rr, sr[rr], vs)
        sfx = jnp.cumsum(v[::-1])[::-1] + above_rows
        lane = jnp.max(jnp.where(sfx >= r, iota, -1))
        bucket = jstar * 16 + lane
        above_cnt = above_rows + jnp.sum(jnp.where(iota > lane, v, 0))
        above_rows_sum = zero_f
        for rr in range(16):
            above_rows_sum = above_rows_sum + jnp.where(jstar < rr, sr[rr], zero_f)
        above_sum_vec = above_rows_sum + jnp.where(iota > lane, vs, 0.0)
        if last:
            eq_cnt = jnp.sum(jnp.where(iota == lane, v, 0))
            eq_sum_vec = jnp.where(iota == lane, vs, 0.0)
        else:
            eq_cnt = jnp.int32(0)
            eq_sum_vec = zero_f
        return bucket, above_cnt, above_sum_vec, eq_cnt, eq_sum_vec

    zero_hall()

    @plsc.parallel_loop(0, _VPT, unroll=8)
    def _(i):
        sl = pl.ds(i * 16, 16)
        b = _bce_bits(pred_v[sl], targ_v[sl])
        bits_v[sl] = b
        key = lax.shift_right_logical(b, 23)
        plsc.addupdate_scatter(hall_v, [key], ones_f)
        plsc.addupdate_scatter(hall_v, [key + 256], plsc.bitcast(b, jnp.float32))

    b0, above_cnt, accS, _, _ = merge_scan(jnp.int32(_K), False)
    r = jnp.int32(_K) - above_cnt
    accC = above_cnt

    def histogram_round(shift, key_mask, prefix_shift, prefix):
        zero_hall()

        @plsc.parallel_loop(0, _VPT, unroll=8)
        def _(i):
            b = bits_v[pl.ds(i * 16, 16)]
            pm = lax.shift_right_logical(b, prefix_shift) == prefix
            key = lax.shift_right_logical(b, shift) & key_mask
            plsc.addupdate_scatter(hall_v, [key], ones_f, mask=pm)
            plsc.addupdate_scatter(hall_v, [key + 256],
                                   plsc.bitcast(b, jnp.float32), mask=pm)

    histogram_round(15, 255, 23, b0)
    b1, above_cnt, above_sum, _, _ = merge_scan(r, False)
    r = r - above_cnt
    accC = accC + above_cnt
    accS = accS + above_sum
    p1 = lax.shift_left(b0, 8) | b1

    histogram_round(7, 255, 15, p1)
    b2, above_cnt, above_sum, _, _ = merge_scan(r, False)
    r = r - above_cnt
    accC = accC + above_cnt
    accS = accS + above_sum
    p2 = lax.shift_left(p1, 8) | b2

    histogram_round(0, 127, 7, p2)
    _, above_cnt, above_sum, eq_cnt, eq_sum = merge_scan(r, True)
    total_c = accC + above_cnt + eq_cnt
    s_vec = accS + above_sum + eq_sum
    total_s = jnp.sum(s_vec)

    @pl.when(sid == 0)
    def _():
        sv = jnp.full((16,), total_s)
        cv = jnp.full((16,), total_c).astype(jnp.float32)
        out_v[:] = sv / (cv + _EPS)
        pltpu.sync_copy(out_v, out_h)


@jax.jit
def kernel(pred, target):
    mesh = plsc.VectorSubcoreMesh(
        core_axis_name="c", subcore_axis_name="s", num_cores=1, num_subcores=16)
    f = pl.kernel(
        _body,
        out_type=(jax.ShapeDtypeStruct((16,), jnp.float32),
                  jax.ShapeDtypeStruct((_NS * _HB,), jnp.float32)),
        mesh=mesh,
        compiler_params=pltpu.CompilerParams(needs_layout_passes=False),
        scratch_types=[
            pltpu.VMEM((_PT,), jnp.float32),
            pltpu.VMEM((_PT,), jnp.float32),
            pltpu.VMEM((_PT,), jnp.int32),
            pltpu.VMEM((_HB,), jnp.float32),
            pltpu.VMEM((_NS * _HB,), jnp.float32),
            pltpu.VMEM((16,), jnp.float32),
        ],
    )
    out, _ = f(pred.reshape(_N), target.reshape(_N))
    return out[0]
